# baseline (speedup 1.0000x reference)
import functools

import jax
import jax.numpy as jnp
from jax import lax
from jax.experimental import pallas as pl
from jax.experimental.pallas import tpu as pltpu
from jax.experimental.pallas import tpu_sc as plsc

BATCH = 16384
NUM_FACTORS = 32
PACK = 128 // NUM_FACTORS
NUM_CORES = 2
NUM_SUBCORES = 16
NUM_WORKERS = NUM_CORES * NUM_SUBCORES
ROWS_PER_WORKER = BATCH // NUM_WORKERS
CHUNK = 128
NUM_CHUNKS = ROWS_PER_WORKER // CHUNK
GROUPS_PER_CHUNK = CHUNK // 16

_mesh = plsc.VectorSubcoreMesh(core_axis_name="c", subcore_axis_name="s")


@functools.partial(
    pl.kernel,
    mesh=_mesh,
    compiler_params=pltpu.CompilerParams(
        needs_layout_passes=False, use_tc_tiling_on_sc=True),
    out_type=jax.ShapeDtypeStruct((BATCH,), jnp.float32),
    scratch_types=[
        pltpu.VMEM((8, CHUNK), jnp.int32),
        pltpu.VMEM((8, CHUNK), jnp.int32),
        pltpu.VMEM((2, CHUNK, 128), jnp.float32),
        pltpu.VMEM((2, CHUNK, 128), jnp.float32),
        pltpu.VMEM((ROWS_PER_WORKER,), jnp.float32),
        pltpu.SemaphoreType.DMA,
    ],
)
def _pmf_sc(ci_hbm, pi_hbm, ct_hbm, pt_hbm, out_hbm,
            idx_v, q_v, c_pack, p_pack, out_v, sem):
    wid = lax.axis_index("s") * NUM_CORES + lax.axis_index("c")
    base = wid * ROWS_PER_WORKER

    pltpu.sync_copy(ci_hbm.at[pl.ds(wid * NUM_CHUNKS, NUM_CHUNKS)],
                    idx_v.at[pl.ds(0, NUM_CHUNKS)])
    pltpu.sync_copy(pi_hbm.at[pl.ds(wid * NUM_CHUNKS, NUM_CHUNKS)],
                    idx_v.at[pl.ds(NUM_CHUNKS, NUM_CHUNKS)])

    for r in range(2 * NUM_CHUNKS):
        for i in range(CHUNK // 16):
            q_v[r, pl.ds(i * 16, 16)] = (
                idx_v[r, pl.ds(i * 16, 16)] >> jnp.int32(2))

    def fire(k):
        buf = k % 2
        return (
            pltpu.async_copy(ct_hbm.at[q_v.at[k]], c_pack.at[buf], sem),
            pltpu.async_copy(pt_hbm.at[q_v.at[NUM_CHUNKS + k]],
                             p_pack.at[buf], sem),
        )

    lanes = lax.iota(jnp.int32, 16)
    pending = fire(0)

    for k in range(NUM_CHUNKS):
        for c in pending:
            c.wait()
        if k + 1 < NUM_CHUNKS:
            pending = fire(k + 1)
        buf = k % 2

        def group_body(g, carry, k=k, buf=buf):
            acc = jnp.zeros((16,), jnp.float32)
            coffv = (idx_v[k, pl.ds(g * 16, 16)] & jnp.int32(3)) * jnp.int32(32)
            poffv = (idx_v[NUM_CHUNKS + k, pl.ds(g * 16, 16)]
                     & jnp.int32(3)) * jnp.int32(32)
            for i in range(16):
                row = g * 16 + i
                coff = coffv[i]
                poff = poffv[i]
                c0 = c_pack[buf, row, pl.ds(coff, 16)]
                c1 = c_pack[buf, row, pl.ds(coff + 16, 16)]
                p0 = p_pack[buf, row, pl.ds(poff, 16)]
                p1 = p_pack[buf, row, pl.ds(poff + 16, 16)]
                s = jnp.sum(c0 * p0 + c1 * p1)
                acc = jnp.where(lanes == i, s, acc)
            out_v[pl.ds(k * CHUNK + g * 16, 16)] = 1.0 / (1.0 + jnp.exp(-acc))
            return carry

        lax.fori_loop(0, GROUPS_PER_CHUNK, group_body, 0)

    pltpu.sync_copy(out_v, out_hbm.at[pl.ds(base, ROWS_PER_WORKER)])


def kernel(climber_indices, problem_indices, climber_table, problem_table):
    ci = climber_indices.astype(jnp.int32).reshape(NUM_WORKERS * NUM_CHUNKS, CHUNK)
    pi = problem_indices.astype(jnp.int32).reshape(NUM_WORKERS * NUM_CHUNKS, CHUNK)
    ct = climber_table.reshape(-1, 128)
    pt = problem_table.reshape(-1, 128)
    return _pmf_sc(ci, pi, ct, pt)
